# baseline (speedup 1.0000x reference)
import functools

import jax
import jax.numpy as jnp
from jax import lax
from jax.experimental import pallas as pl
from jax.experimental.pallas import tpu as pltpu
from jax.experimental.pallas import tpu_sc as plsc

_D = 128
_B = 4096
_H = 50
_NW = 32
_BC = _B // _NW


def _emb_body(textt_hbm, wf_hbm, wt_hbm, out_hbm,
              idx_v, buf, sem_f, sem_t, sem_o):
    wid = lax.axis_index("s") * 2 + lax.axis_index("c")
    b0 = wid * _BC
    pltpu.sync_copy(textt_hbm.at[:, pl.ds(b0, _BC)], idx_v)

    def issue(h, b):
        idxc = idx_v.at[h]
        pltpu.async_copy(wf_hbm.at[idxc], buf.at[b, :, pl.ds(0, _D)],
                         sem_f.at[b])
        pltpu.async_copy(wt_hbm.at[idxc], buf.at[b, :, pl.ds(_D, _D)],
                         sem_t.at[b])

    def wait_gather(h, b):
        idxc = idx_v.at[h]
        pltpu.make_async_copy(wf_hbm.at[idxc], buf.at[b, :, pl.ds(0, _D)],
                              sem_f.at[b]).wait()
        pltpu.make_async_copy(wt_hbm.at[idxc], buf.at[b, :, pl.ds(_D, _D)],
                              sem_t.at[b]).wait()

    def issue_write(h, b):
        pltpu.async_copy(buf.at[b], out_hbm.at[h, pl.ds(b0, _BC), :],
                         sem_o.at[b])

    def wait_write(h, b):
        pltpu.make_async_copy(buf.at[b], out_hbm.at[h, pl.ds(b0, _BC), :],
                              sem_o.at[b]).wait()

    issue(0, 0)

    @pl.loop(0, 48, step=3)
    def _chunk_loop(h0):
        for b in range(3):
            h = h0 + b
            nb = (b + 1) % 3

            @pl.when(h - 2 >= 0)
            def _():
                wait_write(h - 2, nb)

            @pl.when(h + 1 < _H)
            def _():
                issue(h + 1, nb)

            wait_gather(h, b)
            issue_write(h, b)

    for h, b in ((48, 0), (49, 1)):
        wait_write(h - 2, (b + 1) % 3)
        if h + 1 < _H:
            issue(h + 1, (b + 1) % 3)
        wait_gather(h, b)
        issue_write(h, b)
    wait_write(48, 0)
    wait_write(49, 1)


@functools.partial(
    pl.kernel,
    out_type=jax.ShapeDtypeStruct((_H, _B, 2 * _D), jnp.float32),
    mesh=plsc.VectorSubcoreMesh(core_axis_name="c", subcore_axis_name="s"),
    scratch_types=[
        pltpu.VMEM((_H, _BC), jnp.int32),
        pltpu.VMEM((3, _BC, 2 * _D), jnp.float32),
        pltpu.SemaphoreType.DMA((3,)),
        pltpu.SemaphoreType.DMA((3,)),
        pltpu.SemaphoreType.DMA((3,)),
    ],
)
def _emb_lookup(textt_hbm, wf_hbm, wt_hbm, out_hbm,
                idx_v, buf, sem_f, sem_t, sem_o):
    _emb_body(textt_hbm, wf_hbm, wt_hbm, out_hbm,
              idx_v, buf, sem_f, sem_t, sem_o)


def kernel(text, W_frozen, W_train):
    textt = text.T.astype(jnp.int32)
    out = _emb_lookup(textt, W_frozen, W_train)
    return out.transpose(1, 0, 2)


# speedup vs baseline: 1.0127x; 1.0127x over previous
import functools

import jax
import jax.numpy as jnp
from jax import lax
from jax.experimental import pallas as pl
from jax.experimental.pallas import tpu as pltpu
from jax.experimental.pallas import tpu_sc as plsc

_D = 128
_B = 4096
_H = 50
_NW = 32
_BC = _B // _NW


def _emb_body(textt_hbm, wf_hbm, wt_hbm, out_hbm,
              idx_v, buf, sem_f, sem_t, sem_o):
    wid = lax.axis_index("s") * 2 + lax.axis_index("c")
    b0 = wid * _BC
    pltpu.sync_copy(textt_hbm.at[:, pl.ds(b0, _BC)], idx_v)

    def issue(h, b):
        idxc = idx_v.at[h]
        pltpu.async_copy(wf_hbm.at[idxc], buf.at[b, :, pl.ds(0, _D)],
                         sem_f.at[b])
        pltpu.async_copy(wt_hbm.at[idxc], buf.at[b, :, pl.ds(_D, _D)],
                         sem_t.at[b])

    def wait_gather(h, b):
        idxc = idx_v.at[h]
        pltpu.make_async_copy(wf_hbm.at[idxc], buf.at[b, :, pl.ds(0, _D)],
                              sem_f.at[b]).wait()
        pltpu.make_async_copy(wt_hbm.at[idxc], buf.at[b, :, pl.ds(_D, _D)],
                              sem_t.at[b]).wait()

    issue(0, 0)
    issue(1, 1)

    @pl.loop(0, 48, step=3)
    def _chunk_loop(h0):
        for b in range(3):
            h = h0 + b
            nb = (b + 2) % 3

            @pl.when(h + 2 < _H)
            def _():
                issue(h + 2, nb)

            wait_gather(h, b)
            pltpu.sync_copy(buf.at[b], out_hbm.at[h, pl.ds(b0, _BC), :])

    for h, b in ((48, 0), (49, 1)):
        wait_gather(h, b)
        pltpu.sync_copy(buf.at[b], out_hbm.at[h, pl.ds(b0, _BC), :])


@functools.partial(
    pl.kernel,
    out_type=jax.ShapeDtypeStruct((_H, _B, 2 * _D), jnp.float32),
    mesh=plsc.VectorSubcoreMesh(core_axis_name="c", subcore_axis_name="s"),
    scratch_types=[
        pltpu.VMEM((_H, _BC), jnp.int32),
        pltpu.VMEM((3, _BC, 2 * _D), jnp.float32),
        pltpu.SemaphoreType.DMA((3,)),
        pltpu.SemaphoreType.DMA((3,)),
        pltpu.SemaphoreType.DMA((3,)),
    ],
)
def _emb_lookup(textt_hbm, wf_hbm, wt_hbm, out_hbm,
                idx_v, buf, sem_f, sem_t, sem_o):
    _emb_body(textt_hbm, wf_hbm, wt_hbm, out_hbm,
              idx_v, buf, sem_f, sem_t, sem_o)


def kernel(text, W_frozen, W_train):
    textt = text.T.astype(jnp.int32)
    out = _emb_lookup(textt, W_frozen, W_train)
    return out.transpose(1, 0, 2)


# speedup vs baseline: 1.0128x; 1.0000x over previous
import functools

import jax
import jax.numpy as jnp
from jax import lax
from jax.experimental import pallas as pl
from jax.experimental.pallas import tpu as pltpu
from jax.experimental.pallas import tpu_sc as plsc

_D = 128
_B = 4096
_H = 50
_NW = 32
_BC = _B // _NW


def _emb_body(textt_hbm, wf_hbm, wt_hbm, out_hbm,
              idx_v, buf, sem_f, sem_t):
    wid = lax.axis_index("s") * 2 + lax.axis_index("c")
    b0 = wid * _BC
    pltpu.sync_copy(textt_hbm.at[:, pl.ds(b0, _BC)], idx_v)

    def issue(h, b):
        idxc = idx_v.at[h]
        pltpu.async_copy(wf_hbm.at[idxc], buf.at[b, :, pl.ds(0, _D)],
                         sem_f.at[b])
        pltpu.async_copy(wt_hbm.at[idxc], buf.at[b, :, pl.ds(_D, _D)],
                         sem_t.at[b])

    def wait_gather(h, b):
        idxc = idx_v.at[h]
        pltpu.make_async_copy(wf_hbm.at[idxc], buf.at[b, :, pl.ds(0, _D)],
                              sem_f.at[b]).wait()
        pltpu.make_async_copy(wt_hbm.at[idxc], buf.at[b, :, pl.ds(_D, _D)],
                              sem_t.at[b]).wait()

    issue(0, 0)

    @pl.loop(0, _H, step=2)
    def _chunk_loop(h0):
        for b in range(2):
            h = h0 + b

            @pl.when(h + 1 < _H)
            def _():
                issue(h + 1, 1 - b)

            wait_gather(h, b)
            pltpu.sync_copy(buf.at[b], out_hbm.at[h, pl.ds(b0, _BC), :])


@functools.partial(
    pl.kernel,
    out_type=jax.ShapeDtypeStruct((_H, _B, 2 * _D), jnp.float32),
    mesh=plsc.VectorSubcoreMesh(core_axis_name="c", subcore_axis_name="s"),
    scratch_types=[
        pltpu.VMEM((_H, _BC), jnp.int32),
        pltpu.VMEM((2, _BC, 2 * _D), jnp.float32),
        pltpu.SemaphoreType.DMA((2,)),
        pltpu.SemaphoreType.DMA((2,)),
    ],
)
def _emb_lookup(textt_hbm, wf_hbm, wt_hbm, out_hbm,
                idx_v, buf, sem_f, sem_t):
    _emb_body(textt_hbm, wf_hbm, wt_hbm, out_hbm,
              idx_v, buf, sem_f, sem_t)


def kernel(text, W_frozen, W_train):
    textt = text.T.astype(jnp.int32)
    out = _emb_lookup(textt, W_frozen, W_train)
    return out.transpose(1, 0, 2)
